# baseline (speedup 1.0000x reference)
import functools

import jax
import jax.numpy as jnp
from jax import lax
from jax.experimental import pallas as pl
from jax.experimental.pallas import tpu as pltpu
from jax.experimental.pallas import tpu_sc as plsc

D_MODEL = 32
MAX_LEN = 1000000
MAIN_LEN = (MAX_LEN // 128) * 128
TAIL_LEN = 128
PLANE_PAD = MAIN_LEN + TAIL_LEN
NUM_CORES = 2
NUM_SUBCORES = 16
PLANES_PER_CORE = D_MODEL // NUM_CORES
STAGE_SPLIT = 4
STAGE_CHUNK = MAIN_LEN // STAGE_SPLIT


def _stage_copies(tablet_hbm, tail_hbm, plane_sp, psem, d):
  for q in range(STAGE_SPLIT):
    yield pltpu.make_async_copy(
        tablet_hbm.at[d, pl.ds(q * STAGE_CHUNK, STAGE_CHUNK)],
        plane_sp.at[pl.ds(q * STAGE_CHUNK, STAGE_CHUNK)], psem)
  yield pltpu.make_async_copy(
      tail_hbm.at[d], plane_sp.at[pl.ds(MAIN_LEN, TAIL_LEN)], psem)


def _plane_body(xt_hbm, tablet_hbm, tail_hbm, out_hbm,
                idx0, idx1, dest0, dest1, plane_a, plane_b,
                psem_a, psem_b, isem0, isem1, gsem0, gsem1, wsem0, wsem1,
                *, b, l):
  c = lax.axis_index("c")
  s = lax.axis_index("s")
  b_per_tile = b // NUM_SUBCORES
  b0 = s * b_per_tile
  idx_refs = (idx0, idx1)
  dest_refs = (dest0, dest1)
  isems = (isem0, isem1)
  gsems = (gsem0, gsem1)
  wsems = (wsem0, wsem1)
  d_base = c * PLANES_PER_CORE

  def idx_copy(li, buf):
    return pltpu.make_async_copy(xt_hbm.at[li, pl.ds(b0, b_per_tile)],
                                 idx_refs[buf], isems[buf])

  def gather_copy(li, plane_sp, d):
    return pltpu.make_async_copy(plane_sp.at[idx_refs[li % 2]],
                                 dest_refs[li % 2], gsems[li % 2])

  def wb_copy(li, d):
    return pltpu.make_async_copy(
        dest_refs[li % 2],
        out_hbm.at[li, d, pl.ds(b0, b_per_tile)], wsems[li % 2])

  @pl.when(s == 0)
  def _():
    for cp in _stage_copies(tablet_hbm, tail_hbm, plane_a, psem_a, d_base):
      cp.start()

  @pl.loop(0, PLANES_PER_CORE // 2)
  def _pair(jj):
    for k in range(2):
      plane_sp = (plane_a, plane_b)[k]
      psem = (psem_a, psem_b)[k]
      nxt_sp = (plane_b, plane_a)[k]
      nxt_psem = (psem_b, psem_a)[k]
      j = jj * 2 + k
      d = d_base + j

      @pl.when(s == 0)
      def _():
        for cp in _stage_copies(tablet_hbm, tail_hbm, plane_sp, psem, d):
          cp.wait()

      plsc.subcore_barrier()

      if k == 0:
        @pl.when(s == 0)
        def _():
          for cp in _stage_copies(tablet_hbm, tail_hbm, nxt_sp, nxt_psem,
                                  d + 1):
            cp.start()
      else:
        @pl.when(jnp.logical_and(s == 0, jj + 1 < PLANES_PER_CORE // 2))
        def _():
          for cp in _stage_copies(tablet_hbm, tail_hbm, nxt_sp, nxt_psem,
                                  d + 1):
            cp.start()

      idx_copy(0, 0).start()
      idx_copy(1, 1).start()
      for li in range(l):
        ib = li % 2
        idx_copy(li, ib).wait()

        if li >= 2:
          wb_copy(li - 2, d).wait()
        else:
          if k == 0:
            @pl.when(j > 0)
            def _(li=li):
              wb_copy(l - 2 + li, d - 1).wait()
          else:
            wb_copy(l - 2 + li, d - 1).wait()

        gather_copy(li, plane_sp, d).start()

        if li > 0:
          gather_copy(li - 1, plane_sp, d).wait()
          wb_copy(li - 1, d).start()
          if li + 1 < l:
            idx_copy(li + 1, 1 - ib).start()

      gather_copy(l - 1, plane_sp, d).wait()
      wb_copy(l - 1, d).start()

      plsc.subcore_barrier()

  d_last = d_base + PLANES_PER_CORE - 1
  for li in (l - 2, l - 1):
    wb_copy(li, d_last).wait()


def kernel(x, table):
  b, l = x.shape
  xt = x.T.astype(jnp.int32)
  tablet = table.T
  tail = jnp.concatenate(
      [table[MAIN_LEN:], jnp.zeros((TAIL_LEN - (MAX_LEN - MAIN_LEN), D_MODEL),
                                   jnp.float32)], axis=0)
  tail_t = tail.T
  b_per_tile = b // NUM_SUBCORES

  mesh = plsc.VectorSubcoreMesh(core_axis_name="c", subcore_axis_name="s")
  out_t = pl.kernel(
      functools.partial(_plane_body, b=b, l=l),
      out_type=jax.ShapeDtypeStruct((l, D_MODEL, b), jnp.float32),
      mesh=mesh,
      scratch_types=[
          pltpu.VMEM((b_per_tile,), jnp.int32),
          pltpu.VMEM((b_per_tile,), jnp.int32),
          pltpu.VMEM((b_per_tile,), jnp.float32),
          pltpu.VMEM((b_per_tile,), jnp.float32),
          pltpu.VMEM_SHARED((PLANE_PAD,), jnp.float32),
          pltpu.VMEM_SHARED((PLANE_PAD,), jnp.float32),
          pltpu.SemaphoreType.DMA,
          pltpu.SemaphoreType.DMA,
          pltpu.SemaphoreType.DMA,
          pltpu.SemaphoreType.DMA,
          pltpu.SemaphoreType.DMA,
          pltpu.SemaphoreType.DMA,
          pltpu.SemaphoreType.DMA,
          pltpu.SemaphoreType.DMA,
      ],
      compiler_params=pltpu.CompilerParams(use_tc_tiling_on_sc=True),
  )(xt, tablet, tail_t)
  return out_t.transpose(2, 0, 1)


# speedup vs baseline: 1.5157x; 1.5157x over previous
import functools

import jax
import jax.numpy as jnp
from jax import lax
from jax.experimental import pallas as pl
from jax.experimental.pallas import tpu as pltpu
from jax.experimental.pallas import tpu_sc as plsc

D_MODEL = 32
MAX_LEN = 1000000
MAIN_LEN = (MAX_LEN // 128) * 128
TAIL_LEN = 128
PLANE_PAD = MAIN_LEN + TAIL_LEN
NUM_CORES = 2
NUM_SUBCORES = 16
PLANES_PER_CORE = D_MODEL // NUM_CORES
NHALF = 2
NSEG_HALF = 5


def _stage_plane(tablet_hbm, tail_hbm, plane_sp, psem, d):
  pltpu.async_copy(tablet_hbm.at[d, pl.ds(0, MAIN_LEN)],
                   plane_sp.at[pl.ds(0, MAIN_LEN)], psem)
  pltpu.async_copy(tail_hbm.at[d], plane_sp.at[pl.ds(MAIN_LEN, TAIL_LEN)],
                   psem)


def _wait_plane(tablet_hbm, tail_hbm, plane_sp, psem, d):
  pltpu.make_async_copy(tablet_hbm.at[d, pl.ds(0, MAIN_LEN)],
                        plane_sp.at[pl.ds(0, MAIN_LEN)], psem).wait()
  pltpu.make_async_copy(tail_hbm.at[d],
                        plane_sp.at[pl.ds(MAIN_LEN, TAIL_LEN)], psem).wait()


def _plane_body(xt_hbm, tablet_hbm, tail_hbm, out_hbm,
                idx0, idx1, dest0, dest1, plane_sp,
                psem, gsem0, gsem1, wsem0, wsem1, isem,
                *, b, l):
  c = lax.axis_index("c")
  s = lax.axis_index("s")
  b_per_tile = b // NUM_SUBCORES
  b0 = s * b_per_tile
  l_half = l // NHALF
  seg_rows = l_half // NSEG_HALF
  seg = seg_rows * b_per_tile
  nseg = l // seg_rows
  idx_refs = (idx0, idx1)
  dest_refs = (dest0, dest1)
  gsems = (gsem0, gsem1)
  wsems = (wsem0, wsem1)
  d_base = c * PLANES_PER_CORE

  def seg_idx_slice(g):
    return idx_refs[g // NSEG_HALF].at[
        pl.ds((g % NSEG_HALF) * seg, seg)]

  def wb_copies(g, d, dbuf, wsem):
    for r in range(seg_rows):
      li = g * seg_rows + r
      yield pltpu.make_async_copy(
          dbuf.at[pl.ds(r * b_per_tile, b_per_tile)],
          out_hbm.at[li, d, pl.ds(b0, b_per_tile)], wsem)

  for li in range(l):
    pltpu.async_copy(
        xt_hbm.at[li, pl.ds(b0, b_per_tile)],
        idx_refs[li // l_half].at[pl.ds((li % l_half) * b_per_tile,
                                        b_per_tile)], isem)
  for li in range(l):
    pltpu.make_async_copy(
        xt_hbm.at[li, pl.ds(b0, b_per_tile)],
        idx_refs[li // l_half].at[pl.ds((li % l_half) * b_per_tile,
                                        b_per_tile)], isem).wait()

  @pl.when(s == 0)
  def _():
    _stage_plane(tablet_hbm, tail_hbm, plane_sp, psem, d_base)

  @pl.loop(0, PLANES_PER_CORE)
  def _plane(j):
    d = d_base + j

    @pl.when(s == 0)
    def _():
      _wait_plane(tablet_hbm, tail_hbm, plane_sp, psem, d)

    plsc.subcore_barrier()

    for g in range(nseg):
      dbuf = dest_refs[g % 2]
      wsem = wsems[g % 2]
      gsem = gsems[g % 2]

      def drain(g=g, dbuf=dbuf, wsem=wsem):
        for cp in wb_copies(g, d, dbuf, wsem):
          cp.wait()

      if g < 2:
        @pl.when(j > 0)
        def _(drain=drain):
          drain()
      else:
        drain()

      pltpu.async_copy(plane_sp.at[seg_idx_slice(g)], dbuf, gsem)

      if g > 0:
        pg = g - 1
        pltpu.make_async_copy(plane_sp.at[seg_idx_slice(pg)],
                              dest_refs[pg % 2], gsems[pg % 2]).wait()
        for cp in wb_copies(pg, d, dest_refs[pg % 2], wsems[pg % 2]):
          cp.start()

    pg = nseg - 1
    pltpu.make_async_copy(plane_sp.at[seg_idx_slice(pg)],
                          dest_refs[pg % 2], gsems[pg % 2]).wait()
    for cp in wb_copies(pg, d, dest_refs[pg % 2], wsems[pg % 2]):
      cp.start()

    plsc.subcore_barrier()

    @pl.when(jnp.logical_and(s == 0, j + 1 < PLANES_PER_CORE))
    def _():
      _stage_plane(tablet_hbm, tail_hbm, plane_sp, psem, d + 1)

  d_last = d_base + PLANES_PER_CORE - 1
  for g in (nseg - 2, nseg - 1):
    for cp in wb_copies(g, d_last, dest_refs[g % 2], wsems[g % 2]):
      cp.wait()


def kernel(x, table):
  b, l = x.shape
  xt = x.T.astype(jnp.int32)
  tablet = table.T
  tail = jnp.concatenate(
      [table[MAIN_LEN:], jnp.zeros((TAIL_LEN - (MAX_LEN - MAIN_LEN), D_MODEL),
                                   jnp.float32)], axis=0)
  tail_t = tail.T
  b_per_tile = b // NUM_SUBCORES
  l_half = l // NHALF
  seg_elems = (l_half // NSEG_HALF) * b_per_tile

  mesh = plsc.VectorSubcoreMesh(core_axis_name="c", subcore_axis_name="s")
  out_t = pl.kernel(
      functools.partial(_plane_body, b=b, l=l),
      out_type=jax.ShapeDtypeStruct((l, D_MODEL, b), jnp.float32),
      mesh=mesh,
      scratch_types=[
          pltpu.VMEM((l_half * b_per_tile,), jnp.int32),
          pltpu.VMEM((l_half * b_per_tile,), jnp.int32),
          pltpu.VMEM((seg_elems,), jnp.float32),
          pltpu.VMEM((seg_elems,), jnp.float32),
          pltpu.VMEM_SHARED((PLANE_PAD,), jnp.float32),
          pltpu.SemaphoreType.DMA,
          pltpu.SemaphoreType.DMA,
          pltpu.SemaphoreType.DMA,
          pltpu.SemaphoreType.DMA,
          pltpu.SemaphoreType.DMA,
          pltpu.SemaphoreType.DMA,
      ],
      compiler_params=pltpu.CompilerParams(use_tc_tiling_on_sc=True),
  )(xt, tablet, tail_t)
  return out_t.transpose(2, 0, 1)


# speedup vs baseline: 1.5590x; 1.0286x over previous
import functools

import jax
import jax.numpy as jnp
from jax import lax
from jax.experimental import pallas as pl
from jax.experimental.pallas import tpu as pltpu
from jax.experimental.pallas import tpu_sc as plsc

D_MODEL = 32
MAX_LEN = 1000000
MAIN_LEN = (MAX_LEN // 128) * 128
TAIL_LEN = 128
PLANE_PAD = MAIN_LEN + TAIL_LEN
NUM_CORES = 2
NUM_SUBCORES = 16
PLANES_PER_CORE = D_MODEL // NUM_CORES
NHALF = 2
NSEG_HALF = 5


def _stage_plane(tablet_hbm, tail_hbm, plane_sp, psem, d):
  pltpu.async_copy(tablet_hbm.at[d, pl.ds(0, MAIN_LEN)],
                   plane_sp.at[pl.ds(0, MAIN_LEN)], psem)
  pltpu.async_copy(tail_hbm.at[d], plane_sp.at[pl.ds(MAIN_LEN, TAIL_LEN)],
                   psem)


def _wait_plane(tablet_hbm, tail_hbm, plane_sp, psem, d):
  pltpu.make_async_copy(tablet_hbm.at[d, pl.ds(0, MAIN_LEN)],
                        plane_sp.at[pl.ds(0, MAIN_LEN)], psem).wait()
  pltpu.make_async_copy(tail_hbm.at[d],
                        plane_sp.at[pl.ds(MAIN_LEN, TAIL_LEN)], psem).wait()


NDEST = 3


def _plane_body(xt_hbm, tablet_hbm, tail_hbm, out_hbm,
                idx0, idx1, dest0, dest1, dest2, plane_sp,
                psem, gsem0, gsem1, gsem2, wsem0, wsem1, wsem2, isem,
                *, b, l):
  c = lax.axis_index("c")
  s = lax.axis_index("s")
  b_per_tile = b // NUM_SUBCORES
  b0 = s * b_per_tile
  l_half = l // NHALF
  seg_rows = l_half // NSEG_HALF
  seg = seg_rows * b_per_tile
  nseg = l // seg_rows
  idx_refs = (idx0, idx1)
  dest_refs = (dest0, dest1, dest2)
  gsems = (gsem0, gsem1, gsem2)
  wsems = (wsem0, wsem1, wsem2)
  d_base = c * PLANES_PER_CORE

  def seg_idx_slice(g):
    return idx_refs[g // NSEG_HALF].at[
        pl.ds((g % NSEG_HALF) * seg, seg)]

  def wb_copies(g, d, dbuf, wsem):
    for r in range(seg_rows):
      li = g * seg_rows + r
      yield pltpu.make_async_copy(
          dbuf.at[pl.ds(r * b_per_tile, b_per_tile)],
          out_hbm.at[li, d, pl.ds(b0, b_per_tile)], wsem)

  for li in range(l):
    pltpu.async_copy(
        xt_hbm.at[li, pl.ds(b0, b_per_tile)],
        idx_refs[li // l_half].at[pl.ds((li % l_half) * b_per_tile,
                                        b_per_tile)], isem)
  for li in range(l):
    pltpu.make_async_copy(
        xt_hbm.at[li, pl.ds(b0, b_per_tile)],
        idx_refs[li // l_half].at[pl.ds((li % l_half) * b_per_tile,
                                        b_per_tile)], isem).wait()

  @pl.when(s == 0)
  def _():
    _stage_plane(tablet_hbm, tail_hbm, plane_sp, psem, d_base)

  @pl.loop(0, PLANES_PER_CORE)
  def _plane(j):
    d = d_base + j

    @pl.when(s == 0)
    def _():
      _wait_plane(tablet_hbm, tail_hbm, plane_sp, psem, d)

    plsc.subcore_barrier()

    for g in range(nseg):
      dbuf = dest_refs[g % NDEST]
      wsem = wsems[g % NDEST]
      gsem = gsems[g % NDEST]

      def drain(g=g, dbuf=dbuf, wsem=wsem):
        for cp in wb_copies(g, d, dbuf, wsem):
          cp.wait()

      if g < NDEST:
        @pl.when(j > 0)
        def _(drain=drain):
          drain()
      else:
        drain()

      pltpu.async_copy(plane_sp.at[seg_idx_slice(g)], dbuf, gsem)

      if g > 0:
        pg = g - 1
        pltpu.make_async_copy(plane_sp.at[seg_idx_slice(pg)],
                              dest_refs[pg % NDEST], gsems[pg % NDEST]).wait()
        for cp in wb_copies(pg, d, dest_refs[pg % NDEST],
                            wsems[pg % NDEST]):
          cp.start()

    pg = nseg - 1
    pltpu.make_async_copy(plane_sp.at[seg_idx_slice(pg)],
                          dest_refs[pg % NDEST], gsems[pg % NDEST]).wait()
    for cp in wb_copies(pg, d, dest_refs[pg % NDEST], wsems[pg % NDEST]):
      cp.start()

    plsc.subcore_barrier()

    @pl.when(jnp.logical_and(s == 0, j + 1 < PLANES_PER_CORE))
    def _():
      _stage_plane(tablet_hbm, tail_hbm, plane_sp, psem, d + 1)

  d_last = d_base + PLANES_PER_CORE - 1
  for g in range(nseg - NDEST, nseg):
    for cp in wb_copies(g, d_last, dest_refs[g % NDEST], wsems[g % NDEST]):
      cp.wait()


def kernel(x, table):
  b, l = x.shape
  xt = x.T.astype(jnp.int32)
  tablet = table.T
  tail = jnp.concatenate(
      [table[MAIN_LEN:], jnp.zeros((TAIL_LEN - (MAX_LEN - MAIN_LEN), D_MODEL),
                                   jnp.float32)], axis=0)
  tail_t = tail.T
  b_per_tile = b // NUM_SUBCORES
  l_half = l // NHALF
  seg_elems = (l_half // NSEG_HALF) * b_per_tile

  mesh = plsc.VectorSubcoreMesh(core_axis_name="c", subcore_axis_name="s")
  out_t = pl.kernel(
      functools.partial(_plane_body, b=b, l=l),
      out_type=jax.ShapeDtypeStruct((l, D_MODEL, b), jnp.float32),
      mesh=mesh,
      scratch_types=[
          pltpu.VMEM((l_half * b_per_tile,), jnp.int32),
          pltpu.VMEM((l_half * b_per_tile,), jnp.int32),
          pltpu.VMEM((seg_elems,), jnp.float32),
          pltpu.VMEM((seg_elems,), jnp.float32),
          pltpu.VMEM((seg_elems,), jnp.float32),
          pltpu.VMEM_SHARED((PLANE_PAD,), jnp.float32),
          pltpu.SemaphoreType.DMA,
          pltpu.SemaphoreType.DMA,
          pltpu.SemaphoreType.DMA,
          pltpu.SemaphoreType.DMA,
          pltpu.SemaphoreType.DMA,
          pltpu.SemaphoreType.DMA,
          pltpu.SemaphoreType.DMA,
          pltpu.SemaphoreType.DMA,
      ],
      compiler_params=pltpu.CompilerParams(use_tc_tiling_on_sc=True),
  )(xt, tablet, tail_t)
  return out_t.transpose(2, 0, 1)


# speedup vs baseline: 1.5723x; 1.0086x over previous
import functools

import jax
import jax.numpy as jnp
from jax import lax
from jax.experimental import pallas as pl
from jax.experimental.pallas import tpu as pltpu
from jax.experimental.pallas import tpu_sc as plsc

D_MODEL = 32
MAX_LEN = 1000000
MAIN_LEN = (MAX_LEN // 128) * 128
TAIL_LEN = 128
PLANE_PAD = MAIN_LEN + TAIL_LEN
NUM_CORES = 2
NUM_SUBCORES = 16
PLANES_PER_CORE = D_MODEL // NUM_CORES
NHALF = 2
NSEG_HALF = 5


def _stage_plane(tablet_hbm, tail_hbm, plane_sp, psem, d):
  pltpu.async_copy(tablet_hbm.at[d, pl.ds(0, MAIN_LEN)],
                   plane_sp.at[pl.ds(0, MAIN_LEN)], psem)
  pltpu.async_copy(tail_hbm.at[d], plane_sp.at[pl.ds(MAIN_LEN, TAIL_LEN)],
                   psem)


def _wait_plane(tablet_hbm, tail_hbm, plane_sp, psem, d):
  pltpu.make_async_copy(tablet_hbm.at[d, pl.ds(0, MAIN_LEN)],
                        plane_sp.at[pl.ds(0, MAIN_LEN)], psem).wait()
  pltpu.make_async_copy(tail_hbm.at[d],
                        plane_sp.at[pl.ds(MAIN_LEN, TAIL_LEN)], psem).wait()


NDEST = 3


def _plane_body(xt_hbm, tablet_hbm, tail_hbm, out_hbm,
                idx0, idx1, dest0, dest1, dest2, plane_sp,
                psem, gsem0, gsem1, gsem2, wsem0, wsem1, wsem2, isem,
                *, b, l):
  c = lax.axis_index("c")
  s = lax.axis_index("s")
  b_per_tile = b // NUM_SUBCORES
  b0 = s * b_per_tile
  l_half = l // NHALF
  seg_rows = l_half // NSEG_HALF
  seg = seg_rows * b_per_tile
  nseg = l // seg_rows
  idx_refs = (idx0, idx1)
  dest_refs = (dest0, dest1, dest2)
  gsems = (gsem0, gsem1, gsem2)
  wsems = (wsem0, wsem1, wsem2)
  d_base = c * PLANES_PER_CORE

  def seg_idx_slice(g):
    return idx_refs[g // NSEG_HALF].at[
        pl.ds((g % NSEG_HALF) * seg, seg)]

  def wb_copies(g, d, dbuf, wsem):
    for r in range(seg_rows):
      li = g * seg_rows + r
      yield pltpu.make_async_copy(
          dbuf.at[pl.ds(r * b_per_tile, b_per_tile)],
          out_hbm.at[li, d, pl.ds(b0, b_per_tile)], wsem)

  @pl.when(s == 0)
  def _():
    _stage_plane(tablet_hbm, tail_hbm, plane_sp, psem, d_base)

  for li in range(l):
    pltpu.async_copy(
        xt_hbm.at[li, pl.ds(b0, b_per_tile)],
        idx_refs[li // l_half].at[pl.ds((li % l_half) * b_per_tile,
                                        b_per_tile)], isem)
  for li in range(l):
    pltpu.make_async_copy(
        xt_hbm.at[li, pl.ds(b0, b_per_tile)],
        idx_refs[li // l_half].at[pl.ds((li % l_half) * b_per_tile,
                                        b_per_tile)], isem).wait()

  @pl.loop(0, PLANES_PER_CORE)
  def _plane(j):
    d = d_base + j

    @pl.when(s == 0)
    def _():
      _wait_plane(tablet_hbm, tail_hbm, plane_sp, psem, d)

    plsc.subcore_barrier()

    for g in range(nseg):
      dbuf = dest_refs[g % NDEST]
      wsem = wsems[g % NDEST]
      gsem = gsems[g % NDEST]

      def drain(g=g, dbuf=dbuf, wsem=wsem):
        for cp in wb_copies(g, d, dbuf, wsem):
          cp.wait()

      if g < NDEST:
        @pl.when(j > 0)
        def _(drain=drain):
          drain()
      else:
        drain()

      pltpu.async_copy(plane_sp.at[seg_idx_slice(g)], dbuf, gsem)

      if g > 0:
        pg = g - 1
        pltpu.make_async_copy(plane_sp.at[seg_idx_slice(pg)],
                              dest_refs[pg % NDEST], gsems[pg % NDEST]).wait()
        for cp in wb_copies(pg, d, dest_refs[pg % NDEST],
                            wsems[pg % NDEST]):
          cp.start()

    pg = nseg - 1
    pltpu.make_async_copy(plane_sp.at[seg_idx_slice(pg)],
                          dest_refs[pg % NDEST], gsems[pg % NDEST]).wait()
    for cp in wb_copies(pg, d, dest_refs[pg % NDEST], wsems[pg % NDEST]):
      cp.start()

    plsc.subcore_barrier()

    @pl.when(jnp.logical_and(s == 0, j + 1 < PLANES_PER_CORE))
    def _():
      _stage_plane(tablet_hbm, tail_hbm, plane_sp, psem, d + 1)

  d_last = d_base + PLANES_PER_CORE - 1
  for g in range(nseg - NDEST, nseg):
    for cp in wb_copies(g, d_last, dest_refs[g % NDEST], wsems[g % NDEST]):
      cp.wait()


def kernel(x, table):
  b, l = x.shape
  xt = x.T.astype(jnp.int32)
  tablet = table.T
  tail = jnp.concatenate(
      [table[MAIN_LEN:], jnp.zeros((TAIL_LEN - (MAX_LEN - MAIN_LEN), D_MODEL),
                                   jnp.float32)], axis=0)
  tail_t = tail.T
  b_per_tile = b // NUM_SUBCORES
  l_half = l // NHALF
  seg_elems = (l_half // NSEG_HALF) * b_per_tile

  mesh = plsc.VectorSubcoreMesh(core_axis_name="c", subcore_axis_name="s")
  out_t = pl.kernel(
      functools.partial(_plane_body, b=b, l=l),
      out_type=jax.ShapeDtypeStruct((l, D_MODEL, b), jnp.float32),
      mesh=mesh,
      scratch_types=[
          pltpu.VMEM((l_half * b_per_tile,), jnp.int32),
          pltpu.VMEM((l_half * b_per_tile,), jnp.int32),
          pltpu.VMEM((seg_elems,), jnp.float32),
          pltpu.VMEM((seg_elems,), jnp.float32),
          pltpu.VMEM((seg_elems,), jnp.float32),
          pltpu.VMEM_SHARED((PLANE_PAD,), jnp.float32),
          pltpu.SemaphoreType.DMA,
          pltpu.SemaphoreType.DMA,
          pltpu.SemaphoreType.DMA,
          pltpu.SemaphoreType.DMA,
          pltpu.SemaphoreType.DMA,
          pltpu.SemaphoreType.DMA,
          pltpu.SemaphoreType.DMA,
          pltpu.SemaphoreType.DMA,
      ],
      compiler_params=pltpu.CompilerParams(use_tc_tiling_on_sc=True),
  )(xt, tablet, tail_t)
  return out_t.transpose(2, 0, 1)
